# baseline (speedup 1.0000x reference)
import functools

import jax
import jax.numpy as jnp
from jax import lax
from jax.experimental import pallas as pl
from jax.experimental.pallas import tpu as pltpu
from jax.experimental.pallas import tpu_sc as plsc

NC = 2
NS = 16
L = 16

NPAD = 2048
CHUNK = 64


@functools.cache
def _unpool_kernel(B, M, N_in, K, C):
    assert M % L == 0 and K % L == 0
    assert B == 2 * NC
    per_core_rows = 2 * M
    assert per_core_rows % CHUNK == 0
    n_chunks = per_core_rows // CHUNK
    per_tile = -(-n_chunks // NS)
    zero_row = B * N_in
    mesh = plsc.VectorSubcoreMesh(core_axis_name="c", subcore_axis_name="s")

    @functools.partial(
        pl.kernel,
        out_type=jax.ShapeDtypeStruct((B * M, C), jnp.float32),
        mesh=mesh,
        scratch_types=[
            pltpu.VMEM((M,), jnp.int32),
            pltpu.VMEM((2, K), jnp.int32),
            pltpu.VMEM((M,), jnp.int32),
            pltpu.VMEM((M,), jnp.int32),
            pltpu.VMEM_SHARED((2 * M,), jnp.int32),
            pltpu.VMEM((CHUNK,), jnp.int32),
            pltpu.VMEM((CHUNK,), jnp.int32),
            pltpu.VMEM((CHUNK, C), jnp.float32),
            pltpu.VMEM((CHUNK, C), jnp.float32),
            pltpu.SemaphoreType.DMA,
            pltpu.SemaphoreType.DMA,
            pltpu.SemaphoreType.DMA,
            pltpu.SemaphoreType.DMA,
        ],
        compiler_params=pltpu.CompilerParams(needs_layout_passes=False),
    )
    def unpool(mask_hbm, order_hbm, table_hbm, out_hbm,
               mp_v, order_v, g_v, out_v, idx_sh,
               i0, i1, r0, r1, sg0, sg1, sw0, sw1):
        c = lax.axis_index("c")
        s = lax.axis_index("s")

        @pl.when(s < 2)
        def _():
            b = 2 * c + s
            pltpu.sync_copy(mask_hbm.at[b], mp_v)
            pltpu.sync_copy(order_hbm.at[b], order_v)
            boff = b * N_in
            iota = lax.iota(jnp.int32, L)

            def lane_bcast(v, j):
                return v.at[jnp.full((L,), j, jnp.int32)].get(
                    mode="promise_in_bounds"
                )

            def p1(i, carry):
                v = mp_v[pl.ds(i * L, L)]
                cs = plsc.cumsum(v)
                zspread = zero_row + ((iota + i * L) & (NPAD - 1))
                posz = jnp.where(v > 0, cs + carry + (boff - 1), zspread)
                mp_v[pl.ds(i * L, L)] = posz
                g_v[pl.ds(i * L, L)] = iota + i * L
                return carry + lane_bcast(cs, L - 1)

            lax.fori_loop(0, M // L, p1, jnp.zeros((L,), jnp.int32))

            lane0 = iota == 0
            rots = [jnp.where(iota < L - r, iota + r, iota + r - L)
                    for r in range(1, L)]

            def p2(i, _):
                base = K - (i + 1) * L
                fv = lax.rev(order_v[0, pl.ds(base, L)], (0,))
                tv = lax.rev(order_v[1, pl.ds(base, L)], (0,))
                conf = jnp.zeros((L,), jnp.bool_)
                for r in rots:
                    fr = fv.at[r].get(mode="promise_in_bounds")
                    tr = tv.at[r].get(mode="promise_in_bounds")
                    conf = conf | (tv == fr) | (tv == tr)

                def fast():
                    gf = plsc.load_gather(g_v, [fv])
                    plsc.store_scatter(g_v, [tv], gf)

                def slow():
                    for j in range(L):
                        fj = lane_bcast(fv, j)
                        tj = lane_bcast(tv, j)
                        gf = plsc.load_gather(g_v, [fj])
                        plsc.store_scatter(g_v, [tj], gf, mask=lane0)

                lax.cond(jnp.any(conf), slow, fast)
                return 0

            lax.fori_loop(0, K // L, p2, 0)

            def p3(i, _):
                gv = g_v[pl.ds(i * L, L)]
                out_v[pl.ds(i * L, L)] = plsc.load_gather(mp_v, [gv])
                return 0

            lax.fori_loop(0, M // L, p3, 0)
            pltpu.sync_copy(out_v, idx_sh.at[pl.ds(s * M, M)])

        plsc.subcore_barrier()

        idx_v = [i0, i1]
        rows_v = [r0, r1]
        sg = [sg0, sg1]
        sw = [sw0, sw1]
        row0 = c * per_core_rows

        def cid(j):
            return s + NS * j

        def gather_desc(sl):
            return pltpu.make_async_copy(table_hbm.at[idx_v[sl]], rows_v[sl],
                                         sg[sl])

        def wb_desc(sl, j):
            return pltpu.make_async_copy(
                rows_v[sl], out_hbm.at[pl.ds(row0 + cid(j) * CHUNK, CHUNK)],
                sw[sl],
            )

        for j in range(per_tile + 2):
            sl = j % 2
            if j >= 2:
                @pl.when(cid(j - 2) < n_chunks)
                def _(j=j, sl=sl):
                    wb_desc(sl, j - 2).wait()
            if j < per_tile:
                @pl.when(cid(j) < n_chunks)
                def _(j=j, sl=sl):
                    pltpu.sync_copy(idx_sh.at[pl.ds(cid(j) * CHUNK, CHUNK)],
                                    idx_v[sl])
                    gather_desc(sl).start()
            if 1 <= j <= per_tile:
                psl = (j - 1) % 2

                @pl.when(cid(j - 1) < n_chunks)
                def _(j=j, psl=psl):
                    gather_desc(psl).wait()
                    wb_desc(psl, j - 1).start()

    return unpool


def kernel(images, mask, order):
    B, N_in, C = images.shape
    M = mask.shape[1]
    K = order.shape[2]

    table = jnp.concatenate(
        [images.reshape(B * N_in, C), jnp.zeros((NPAD, C), images.dtype)], axis=0
    )
    out = _unpool_kernel(B, M, N_in, K, C)(
        mask.astype(jnp.int32), order.astype(jnp.int32), table
    )
    return out.reshape(B, M, C)
